# baseline (speedup 1.0000x reference)
import functools
import numpy as np
import jax
import jax.numpy as jnp
from jax.experimental import pallas as pl
from jax.experimental.pallas import tpu as pltpu

f32 = jnp.float32
i32 = jnp.int32
HIGH = jax.lax.Precision.HIGHEST
BIGD = 1e30
BIGI = 2**30


def _dot(a, b):
    return jax.lax.dot_general(a, b, (((1,), (0,)), ((), ())),
                               precision=HIGH, preferred_element_type=f32)


def _dot_bf(a, b):
    return jax.lax.dot_general(a.astype(jnp.bfloat16), b.astype(jnp.bfloat16),
                               (((1,), (0,)), ((), ())),
                               preferred_element_type=f32)


def _bfify(v):
    return v.astype(jnp.bfloat16).astype(f32)


def _knn_body(nk, npar, bq_blk, k, delta, posq_ref, bq_ref, posT_ref,
              posk_ref, bk_ref, rel_ref, mloc_ref, idxl_ref, mpar_ref=None):
    q = posq_ref[...]
    bq = bq_ref[...]
    pT = posT_ref[...]
    pk = posk_ref[...]
    bk = bk_ref[...]
    qq = jnp.sum(q * q, axis=1, keepdims=True)
    kk = jnp.sum(pT * pT, axis=0, keepdims=True)
    d = qq + kk - 2.0 * _dot_bf(q, pT)
    d = jnp.maximum(d, 0.0)
    d = jnp.where(bq != bk, BIGD, d)
    lid = jax.lax.broadcasted_iota(i32, (1, nk), 1)
    mloc = jnp.zeros((bq_blk, nk), f32)
    mpar = jnp.zeros((bq_blk, npar), f32) if mpar_ref is not None else None
    idxl = []
    for t in range(k):
        sell = jnp.argmin(d, axis=1, keepdims=True).astype(i32)
        o = lid == sell
        of = o.astype(f32)
        rel_ref[t] = _dot(of, pk) - q
        mloc = mloc + of
        idxl.append(sell)
        if mpar_ref is not None:
            selp = jnp.round(sell.astype(f32) * delta).astype(i32)
            piota = jax.lax.broadcasted_iota(i32, (1, npar), 1)
            mpar = mpar + (piota == selp).astype(f32)
        d = jnp.where(o, jnp.inf, d)
    mloc_ref[...] = mloc
    idxl_ref[...] = jnp.concatenate(idxl, axis=1)
    if mpar_ref is not None:
        mpar_ref[...] = mpar


def _knn(pos_q, b_q, pos_k, b_k, bq_blk, par_ids=None, npar=0, k=16):
    nq = pos_q.shape[0]
    nk = pos_k.shape[0]
    grid = nq // bq_blk
    expand = par_ids is not None
    delta = float(np.float32(npar - 1) / np.float32(nk - 1)) if expand else 0.0
    body = functools.partial(_knn_body, nk, npar, bq_blk, k, delta)
    full = lambda shp: pl.BlockSpec(shp, lambda i: (0,) * len(shp))
    in_specs = [
        pl.BlockSpec((bq_blk, 3), lambda i: (i, 0)),
        pl.BlockSpec((bq_blk, 1), lambda i: (i, 0)),
        full((3, nk)), full((nk, 3)), full((1, nk)),
    ]
    args = [pos_q, b_q.reshape(nq, 1), pos_k.T, pos_k, b_k.reshape(1, nk)]
    out_shape = [
        jax.ShapeDtypeStruct((k, nq, 3), f32),
        jax.ShapeDtypeStruct((nq, nk), f32),
        jax.ShapeDtypeStruct((nq, k), i32),
    ]
    out_specs = [
        pl.BlockSpec((k, bq_blk, 3), lambda i: (0, i, 0)),
        pl.BlockSpec((bq_blk, nk), lambda i: (i, 0)),
        pl.BlockSpec((bq_blk, k), lambda i: (i, 0)),
    ]
    if expand:
        out_shape.append(jax.ShapeDtypeStruct((nq, npar), f32))
        out_specs.append(pl.BlockSpec((bq_blk, npar), lambda i: (i, 0)))
    return pl.pallas_call(body, grid=(grid,), in_specs=in_specs,
                          out_specs=out_specs, out_shape=out_shape)(*args)


def _interp_body(nk, bq_blk, k, posq_ref, bq_ref, posT_ref, bk_ref, w_ref):
    q = posq_ref[...]
    bq = bq_ref[...]
    pT = posT_ref[...]
    bk = bk_ref[...]
    qq = jnp.sum(q * q, axis=1, keepdims=True)
    kk = jnp.sum(pT * pT, axis=0, keepdims=True)
    d = qq + kk - 2.0 * _dot_bf(q, pT)
    d = jnp.maximum(d, 0.0)
    d = jnp.where(bq != bk, BIGD, d)
    lid = jax.lax.broadcasted_iota(i32, (1, nk), 1)
    wacc = jnp.zeros((bq_blk, nk), f32)
    wsum = jnp.zeros((bq_blk, 1), f32)
    for _ in range(k):
        m = jnp.min(d, axis=1, keepdims=True)
        sell = jnp.argmin(d, axis=1, keepdims=True).astype(i32)
        o = lid == sell
        wt = 1.0 / (m + 1e-8)
        wacc = wacc + wt * o.astype(f32)
        wsum = wsum + wt
        d = jnp.where(o, jnp.inf, d)
    w_ref[...] = wacc / wsum


def _interp_w(pos_q, b_q, pos_k, b_k, bq_blk, k=3):
    nq = pos_q.shape[0]
    nk = pos_k.shape[0]
    body = functools.partial(_interp_body, nk, bq_blk, k)
    full = lambda shp: pl.BlockSpec(shp, lambda i: (0,) * len(shp))
    return pl.pallas_call(
        body, grid=(nq // bq_blk,),
        in_specs=[pl.BlockSpec((bq_blk, 3), lambda i: (i, 0)),
                  pl.BlockSpec((bq_blk, 1), lambda i: (i, 0)),
                  full((3, nk)), full((1, nk))],
        out_specs=pl.BlockSpec((bq_blk, nk), lambda i: (i, 0)),
        out_shape=jax.ShapeDtypeStruct((nq, nk), f32),
    )(pos_q, b_q.reshape(nq, 1), pos_k.T, b_k.reshape(1, nk))


def _pconv_body(nxs, has_glob, blk, k, *refs):
    it = iter(refs)
    rel_ref = next(it)
    wp1 = next(it)[...]
    wp2 = next(it)[...]
    wfh = next(it)[...]
    m_ref = next(it) if (nxs or has_glob) else None
    hsum = jnp.zeros((blk, wp2.shape[1]), f32)
    for t in range(k):
        h = jnp.maximum(_dot_bf(rel_ref[t], wp1), 0.0)
        h = jnp.maximum(_dot_bf(h, wp2), 0.0)
        hsum = hsum + _bfify(h)
    acc = _dot(hsum * (1.0 / k), _bfify(wfh))
    mb = m_ref[...] if m_ref is not None else None
    for _ in range(nxs):
        x = next(it)[...]
        wfx = next(it)[...]
        acc = acc + _dot(_dot(mb, _bfify(x)) * (1.0 / k), _bfify(wfx))
    if has_glob:
        bk = next(it)[...]
        glob = next(it)[...]
        wfg = next(it)[...]
        ohk = (bk == jax.lax.broadcasted_iota(i32, (1, 4), 1)).astype(f32)
        gm = _dot(mb, ohk) * (1.0 / k)
        acc = acc + _dot(_dot(gm, _bfify(glob)), _bfify(wfg))
    out_ref = next(it)
    out_ref[...] = jnp.maximum(acc, 0.0)


def _pick_blk(n, target):
    for b in range(min(target, n), 7, -8):
        if n % b == 0:
            return b
    return n


def _pconv(rel, wp1, wp2, wfh, m=None, xs=(), glob_piece=None, blk=256, k=16):
    n = rel.shape[1]
    blk = _pick_blk(n, blk)
    cout = wfh.shape[1]
    full = lambda shp: pl.BlockSpec(shp, lambda i: (0,) * len(shp))
    in_specs = [pl.BlockSpec((k, blk, 3), lambda i: (0, i, 0)),
                full(wp1.shape), full(wp2.shape), full(wfh.shape)]
    args = [rel, wp1, wp2, wfh]
    if m is not None:
        ncols = m.shape[1]
        in_specs.append(pl.BlockSpec((blk, ncols), lambda i: (i, 0)))
        args.append(m)
    for x, wfx in xs:
        in_specs += [full(x.shape), full(wfx.shape)]
        args += [x, wfx]
    if glob_piece is not None:
        bk_col, glob, wfg = glob_piece
        in_specs += [full((bk_col.shape[0], 1)), full(glob.shape),
                     full(wfg.shape)]
        args += [bk_col.reshape(-1, 1), glob, wfg]
    body = functools.partial(_pconv_body, len(xs), glob_piece is not None,
                             blk, k)
    return pl.pallas_call(
        body, grid=(n // blk,), in_specs=in_specs,
        out_specs=pl.BlockSpec((blk, cout), lambda i: (i, 0)),
        out_shape=jax.ShapeDtypeStruct((n, cout), f32),
    )(*args)


def _pool_body(f3_ref, b2_ref, w1_ref, b1_ref, w2_ref, b2w_ref, out_ref):
    f3 = f3_ref[...]
    b2 = b2_ref[...]
    ohT = (jax.lax.broadcasted_iota(i32, (4, 1), 0) == b2).astype(f32)
    seg = _dot(ohT, f3)
    cnt = jnp.sum(ohT, axis=1, keepdims=True)
    g = seg / (cnt + 1e-9)
    g = jnp.maximum(_dot_bf(g, w1_ref[...]) + b1_ref[...], 0.0)
    g = jnp.maximum(_dot_bf(g, w2_ref[...]) + b2w_ref[...], 0.0)
    out_ref[...] = g


def _pool(f3, b2, w1, b1, w2, b2w):
    n2 = f3.shape[0]
    return pl.pallas_call(
        _pool_body,
        out_shape=jax.ShapeDtypeStruct((4, w2.shape[1]), f32),
    )(f3, b2.reshape(1, n2), w1, b1.reshape(1, -1), w2, b2w.reshape(1, -1))


def _mm_body(w_ref, x_ref, out_ref):
    out_ref[...] = _dot(w_ref[...], x_ref[...])


def _matmul(w, x, blk=512):
    n, nk = w.shape
    blk = _pick_blk(n, blk)
    c = x.shape[1]
    return pl.pallas_call(
        _mm_body, grid=(n // blk,),
        in_specs=[pl.BlockSpec((blk, nk), lambda i: (i, 0)),
                  pl.BlockSpec((nk, c), lambda i: (0, 0))],
        out_specs=pl.BlockSpec((blk, c), lambda i: (i, 0)),
        out_shape=jax.ShapeDtypeStruct((n, c), f32),
    )(w, x)


def _head_body(n, wh1_ref, gamma_ref, beta_ref, wh2_ref, bh2_ref, u_ref,
               out_ref, hbuf):
    wh1 = wh1_ref[...]
    cr = 512
    s = jnp.zeros((1, 128), f32)
    s2 = jnp.zeros((1, 128), f32)
    for i in range(n // cr):
        h = _dot_bf(u_ref[pl.ds(i * cr, cr), :], wh1)
        hbuf[pl.ds(i * cr, cr), :] = h
        s = s + jnp.sum(h, axis=0, keepdims=True)
        s2 = s2 + jnp.sum(h * h, axis=0, keepdims=True)
    mu = s * (1.0 / n)
    var = s2 * (1.0 / n) - mu * mu
    rstd = jax.lax.rsqrt(var + 1e-5)
    scale = rstd * gamma_ref[...]
    wh2 = wh2_ref[...]
    bh2 = bh2_ref[...]
    for i in range(n // cr):
        h = (hbuf[pl.ds(i * cr, cr), :] - mu) * scale + beta_ref[...]
        h = jnp.maximum(h, 0.0)
        out_ref[pl.ds(i * cr, cr), :] = _dot_bf(h, wh2) + bh2


def _head(u, wh1, gamma, beta, wh2, bh2):
    n = u.shape[0]
    return pl.pallas_call(
        functools.partial(_head_body, n),
        out_shape=jax.ShapeDtypeStruct((n, 13), f32),
        scratch_shapes=[pltpu.VMEM((n, 128), f32)],
    )(wh1, gamma.reshape(1, 128), beta.reshape(1, 128), wh2,
      bh2.reshape(1, 13), u)


def _subidx(n, ratio):
    m = int(n * ratio)
    return np.round(np.linspace(0, n - 1, m)).astype(np.int32)


def kernel(pos, batch, Wp1_d0, Wp2_d0, Wf_d0, Wp1_d1, Wp2_d1, Wf_d1,
           Wp1_d2, Wp2_d2, Wf_d2, Wp1_d3, Wp2_d3, Wf_d3,
           Wp1_u0, Wp2_u0, Wf_u0, Wp1_u1, Wp2_u1, Wf_u1,
           Wp1_u2, Wp2_u2, Wf_u2, Wp1_u3, Wp2_u3, Wf_u3,
           W1, b1, W2, b2, Wh1, gamma, beta, Wh2, bh2):
    n0 = pos.shape[0]
    i1 = _subidx(n0, 0.375)
    n1 = i1.shape[0]
    i2 = _subidx(n1, 0.375)
    pos1, b1v = pos[i1], batch[i1]
    pos2, b2v = pos1[i2], b1v[i2]

    rel0, m0s, idx0 = _knn(pos, batch, pos, batch, 128)
    rel1, m1s, idx1, m1e = _knn(pos1, b1v, pos1, b1v, 128,
                                par_ids=i1, npar=n0)
    rel2, m2s, idx2, m2e = _knn(pos2, b2v, pos2, b2v, 192,
                                par_ids=i2, npar=n1)
    f0 = _pconv(rel0, Wp1_d0, Wp2_d0, Wf_d0)
    f1 = _pconv(rel1, Wp1_d1, Wp2_d1, Wf_d1[:32], m=m1e,
                xs=[(f0, Wf_d1[32:])])
    f2 = _pconv(rel2, Wp1_d2, Wp2_d2, Wf_d2[:32], m=m2e,
                xs=[(f1, Wf_d2[32:])])
    f3 = _pconv(rel2, Wp1_d3, Wp2_d3, Wf_d3[:32], m=m2s,
                xs=[(f2, Wf_d3[32:])])
    glob = _pool(f3, b2v, W1, b1, W2, b2)
    u0 = _pconv(rel2, Wp1_u0, Wp2_u0, Wf_u0[:256], m=m2s,
                xs=[(f3, Wf_u0[384:768]), (f3, Wf_u0[768:])],
                glob_piece=(b2v, glob, Wf_u0[256:384]))
    u1 = _pconv(rel2, Wp1_u1, Wp2_u1, Wf_u1[:256], m=m2s,
                xs=[(u0, Wf_u1[256:640]), (f2, Wf_u1[640:])])
    wi1 = _interp_w(pos1, b1v, pos2, b2v, 128)
    v1 = _matmul(wi1, u1)
    u2 = _pconv(rel1, Wp1_u2, Wp2_u2, Wf_u2[:256], m=m1s,
                xs=[(v1, Wf_u2[256:448]), (f1, Wf_u2[448:])])
    wi0 = _interp_w(pos, batch, pos1, b1v, 128)
    v0 = _matmul(wi0, u2)
    u3 = _pconv(rel0, Wp1_u3, Wp2_u3, Wf_u3[:256], m=m0s,
                xs=[(v0, Wf_u3[256:352]), (f0, Wf_u3[352:])])
    return _head(u3, Wh1, gamma, beta, Wh2, bh2)


# speedup vs baseline: 1.5516x; 1.5516x over previous
import functools
import numpy as np
import jax
import jax.numpy as jnp
from jax.experimental import pallas as pl
from jax.experimental.pallas import tpu as pltpu

f32 = jnp.float32
i32 = jnp.int32
HIGH = jax.lax.Precision.HIGHEST
BIGD = 1e30
BIGI = 2**30


def _dot(a, b):
    return jax.lax.dot_general(a, b, (((1,), (0,)), ((), ())),
                               precision=HIGH, preferred_element_type=f32)


def _dot_bf(a, b):
    return jax.lax.dot_general(a.astype(jnp.bfloat16), b.astype(jnp.bfloat16),
                               (((1,), (0,)), ((), ())),
                               preferred_element_type=f32)


def _bfify(v):
    return v.astype(jnp.bfloat16).astype(f32)


def _knn_body(nk, npar, bq_blk, k, delta, posq_ref, bq_ref, posT_ref,
              posk_ref, bk_ref, rel_ref, mloc_ref, idxl_ref, mpar_ref=None):
    q = posq_ref[...]
    bq = bq_ref[...]
    pT = posT_ref[...]
    pk = posk_ref[...]
    bk = bk_ref[...]
    qq = jnp.sum(q * q, axis=1, keepdims=True)
    kk = jnp.sum(pT * pT, axis=0, keepdims=True)
    d = qq + kk - 2.0 * _dot_bf(q, pT)
    d = jnp.maximum(d, 0.0)
    d = jnp.where(bq != bk, BIGD, d)
    lid = jax.lax.broadcasted_iota(i32, (1, nk), 1)
    mloc = jnp.zeros((bq_blk, nk), f32)
    mpar = jnp.zeros((bq_blk, npar), f32) if mpar_ref is not None else None
    idxl = []
    px = pT[0:1, :]
    py = pT[1:2, :]
    pz = pT[2:3, :]
    for t in range(k):
        sell = jnp.argmin(d, axis=1, keepdims=True).astype(i32)
        o = lid == sell
        rx = jnp.min(jnp.where(o, px, BIGD), axis=1, keepdims=True)
        ry = jnp.min(jnp.where(o, py, BIGD), axis=1, keepdims=True)
        rz = jnp.min(jnp.where(o, pz, BIGD), axis=1, keepdims=True)
        rel_ref[t] = jnp.concatenate([rx, ry, rz], axis=1) - q
        mloc = mloc + o.astype(f32)
        idxl.append(sell)
        if mpar_ref is not None:
            selp = jnp.round(sell.astype(f32) * delta).astype(i32)
            piota = jax.lax.broadcasted_iota(i32, (1, npar), 1)
            mpar = mpar + (piota == selp).astype(f32)
        d = jnp.where(o, jnp.inf, d)
    mloc_ref[...] = mloc
    idxl_ref[...] = jnp.concatenate(idxl, axis=1)
    if mpar_ref is not None:
        mpar_ref[...] = mpar


def _knn(pos_q, b_q, pos_k, b_k, bq_blk, par_ids=None, npar=0, k=16):
    nq = pos_q.shape[0]
    nk = pos_k.shape[0]
    grid = nq // bq_blk
    expand = par_ids is not None
    delta = float(np.float32(npar - 1) / np.float32(nk - 1)) if expand else 0.0
    body = functools.partial(_knn_body, nk, npar, bq_blk, k, delta)
    full = lambda shp: pl.BlockSpec(shp, lambda i: (0,) * len(shp))
    in_specs = [
        pl.BlockSpec((bq_blk, 3), lambda i: (i, 0)),
        pl.BlockSpec((bq_blk, 1), lambda i: (i, 0)),
        full((3, nk)), full((nk, 3)), full((1, nk)),
    ]
    args = [pos_q, b_q.reshape(nq, 1), pos_k.T, pos_k, b_k.reshape(1, nk)]
    out_shape = [
        jax.ShapeDtypeStruct((k, nq, 3), f32),
        jax.ShapeDtypeStruct((nq, nk), f32),
        jax.ShapeDtypeStruct((nq, k), i32),
    ]
    out_specs = [
        pl.BlockSpec((k, bq_blk, 3), lambda i: (0, i, 0)),
        pl.BlockSpec((bq_blk, nk), lambda i: (i, 0)),
        pl.BlockSpec((bq_blk, k), lambda i: (i, 0)),
    ]
    if expand:
        out_shape.append(jax.ShapeDtypeStruct((nq, npar), f32))
        out_specs.append(pl.BlockSpec((bq_blk, npar), lambda i: (i, 0)))
    return pl.pallas_call(body, grid=(grid,), in_specs=in_specs,
                          out_specs=out_specs, out_shape=out_shape)(*args)


def _interp_body(nk, bq_blk, k, posq_ref, bq_ref, posT_ref, bk_ref, w_ref):
    q = posq_ref[...]
    bq = bq_ref[...]
    pT = posT_ref[...]
    bk = bk_ref[...]
    qq = jnp.sum(q * q, axis=1, keepdims=True)
    kk = jnp.sum(pT * pT, axis=0, keepdims=True)
    d = qq + kk - 2.0 * _dot_bf(q, pT)
    d = jnp.maximum(d, 0.0)
    d = jnp.where(bq != bk, BIGD, d)
    lid = jax.lax.broadcasted_iota(i32, (1, nk), 1)
    wacc = jnp.zeros((bq_blk, nk), f32)
    wsum = jnp.zeros((bq_blk, 1), f32)
    for _ in range(k):
        m = jnp.min(d, axis=1, keepdims=True)
        sell = jnp.argmin(d, axis=1, keepdims=True).astype(i32)
        o = lid == sell
        wt = 1.0 / (m + 1e-8)
        wacc = wacc + wt * o.astype(f32)
        wsum = wsum + wt
        d = jnp.where(o, jnp.inf, d)
    w_ref[...] = wacc / wsum


def _interp_w(pos_q, b_q, pos_k, b_k, bq_blk, k=3):
    nq = pos_q.shape[0]
    nk = pos_k.shape[0]
    body = functools.partial(_interp_body, nk, bq_blk, k)
    full = lambda shp: pl.BlockSpec(shp, lambda i: (0,) * len(shp))
    return pl.pallas_call(
        body, grid=(nq // bq_blk,),
        in_specs=[pl.BlockSpec((bq_blk, 3), lambda i: (i, 0)),
                  pl.BlockSpec((bq_blk, 1), lambda i: (i, 0)),
                  full((3, nk)), full((1, nk))],
        out_specs=pl.BlockSpec((bq_blk, nk), lambda i: (i, 0)),
        out_shape=jax.ShapeDtypeStruct((nq, nk), f32),
    )(pos_q, b_q.reshape(nq, 1), pos_k.T, b_k.reshape(1, nk))


def _pconv_body(nxs, has_glob, blk, k, *refs):
    it = iter(refs)
    rel_ref = next(it)
    wp1 = next(it)[...]
    wp2 = next(it)[...]
    wfh = next(it)[...]
    m_ref = next(it) if (nxs or has_glob) else None
    hsum = jnp.zeros((blk, wp2.shape[1]), f32)
    for t in range(k):
        h = jnp.maximum(_dot_bf(rel_ref[t], wp1), 0.0)
        h = jnp.maximum(_dot_bf(h, wp2), 0.0)
        hsum = hsum + _bfify(h)
    acc = _dot(hsum * (1.0 / k), _bfify(wfh))
    mb = m_ref[...] if m_ref is not None else None
    for _ in range(nxs):
        x = next(it)[...]
        wfx = next(it)[...]
        acc = acc + _dot(_dot(mb, _bfify(x)) * (1.0 / k), _bfify(wfx))
    if has_glob:
        bk = next(it)[...]
        glob = next(it)[...]
        wfg = next(it)[...]
        ohk = (bk == jax.lax.broadcasted_iota(i32, (1, 4), 1)).astype(f32)
        gm = _dot(mb, ohk) * (1.0 / k)
        acc = acc + _dot(_dot(gm, _bfify(glob)), _bfify(wfg))
    out_ref = next(it)
    out_ref[...] = jnp.maximum(acc, 0.0)


def _pick_blk(n, target):
    for b in range(min(target, n), 7, -8):
        if n % b == 0:
            return b
    return n


def _pconv(rel, wp1, wp2, wfh, m=None, xs=(), glob_piece=None, blk=256, k=16):
    n = rel.shape[1]
    blk = _pick_blk(n, blk)
    cout = wfh.shape[1]
    full = lambda shp: pl.BlockSpec(shp, lambda i: (0,) * len(shp))
    in_specs = [pl.BlockSpec((k, blk, 3), lambda i: (0, i, 0)),
                full(wp1.shape), full(wp2.shape), full(wfh.shape)]
    args = [rel, wp1, wp2, wfh]
    if m is not None:
        ncols = m.shape[1]
        in_specs.append(pl.BlockSpec((blk, ncols), lambda i: (i, 0)))
        args.append(m)
    for x, wfx in xs:
        in_specs += [full(x.shape), full(wfx.shape)]
        args += [x, wfx]
    if glob_piece is not None:
        bk_col, glob, wfg = glob_piece
        in_specs += [full((bk_col.shape[0], 1)), full(glob.shape),
                     full(wfg.shape)]
        args += [bk_col.reshape(-1, 1), glob, wfg]
    body = functools.partial(_pconv_body, len(xs), glob_piece is not None,
                             blk, k)
    return pl.pallas_call(
        body, grid=(n // blk,), in_specs=in_specs,
        out_specs=pl.BlockSpec((blk, cout), lambda i: (i, 0)),
        out_shape=jax.ShapeDtypeStruct((n, cout), f32),
    )(*args)


def _pool_body(f3_ref, b2_ref, w1_ref, b1_ref, w2_ref, b2w_ref, out_ref):
    f3 = f3_ref[...]
    b2 = b2_ref[...]
    ohT = (jax.lax.broadcasted_iota(i32, (4, 1), 0) == b2).astype(f32)
    seg = _dot(ohT, f3)
    cnt = jnp.sum(ohT, axis=1, keepdims=True)
    g = seg / (cnt + 1e-9)
    g = jnp.maximum(_dot_bf(g, w1_ref[...]) + b1_ref[...], 0.0)
    g = jnp.maximum(_dot_bf(g, w2_ref[...]) + b2w_ref[...], 0.0)
    out_ref[...] = g


def _pool(f3, b2, w1, b1, w2, b2w):
    n2 = f3.shape[0]
    return pl.pallas_call(
        _pool_body,
        out_shape=jax.ShapeDtypeStruct((4, w2.shape[1]), f32),
    )(f3, b2.reshape(1, n2), w1, b1.reshape(1, -1), w2, b2w.reshape(1, -1))


def _mm_body(w_ref, x_ref, out_ref):
    out_ref[...] = _dot(w_ref[...], x_ref[...])


def _matmul(w, x, blk=512):
    n, nk = w.shape
    blk = _pick_blk(n, blk)
    c = x.shape[1]
    return pl.pallas_call(
        _mm_body, grid=(n // blk,),
        in_specs=[pl.BlockSpec((blk, nk), lambda i: (i, 0)),
                  pl.BlockSpec((nk, c), lambda i: (0, 0))],
        out_specs=pl.BlockSpec((blk, c), lambda i: (i, 0)),
        out_shape=jax.ShapeDtypeStruct((n, c), f32),
    )(w, x)


def _head_body(n, wh1_ref, gamma_ref, beta_ref, wh2_ref, bh2_ref, u_ref,
               out_ref, hbuf):
    wh1 = wh1_ref[...]
    cr = 512
    s = jnp.zeros((1, 128), f32)
    s2 = jnp.zeros((1, 128), f32)
    for i in range(n // cr):
        h = _dot_bf(u_ref[pl.ds(i * cr, cr), :], wh1)
        hbuf[pl.ds(i * cr, cr), :] = h
        s = s + jnp.sum(h, axis=0, keepdims=True)
        s2 = s2 + jnp.sum(h * h, axis=0, keepdims=True)
    mu = s * (1.0 / n)
    var = s2 * (1.0 / n) - mu * mu
    rstd = jax.lax.rsqrt(var + 1e-5)
    scale = rstd * gamma_ref[...]
    wh2 = wh2_ref[...]
    bh2 = bh2_ref[...]
    for i in range(n // cr):
        h = (hbuf[pl.ds(i * cr, cr), :] - mu) * scale + beta_ref[...]
        h = jnp.maximum(h, 0.0)
        out_ref[pl.ds(i * cr, cr), :] = _dot_bf(h, wh2) + bh2


def _head(u, wh1, gamma, beta, wh2, bh2):
    n = u.shape[0]
    return pl.pallas_call(
        functools.partial(_head_body, n),
        out_shape=jax.ShapeDtypeStruct((n, 13), f32),
        scratch_shapes=[pltpu.VMEM((n, 128), f32)],
    )(wh1, gamma.reshape(1, 128), beta.reshape(1, 128), wh2,
      bh2.reshape(1, 13), u)


def _subidx(n, ratio):
    m = int(n * ratio)
    return np.round(np.linspace(0, n - 1, m)).astype(np.int32)


def kernel(pos, batch, Wp1_d0, Wp2_d0, Wf_d0, Wp1_d1, Wp2_d1, Wf_d1,
           Wp1_d2, Wp2_d2, Wf_d2, Wp1_d3, Wp2_d3, Wf_d3,
           Wp1_u0, Wp2_u0, Wf_u0, Wp1_u1, Wp2_u1, Wf_u1,
           Wp1_u2, Wp2_u2, Wf_u2, Wp1_u3, Wp2_u3, Wf_u3,
           W1, b1, W2, b2, Wh1, gamma, beta, Wh2, bh2):
    n0 = pos.shape[0]
    i1 = _subidx(n0, 0.375)
    n1 = i1.shape[0]
    i2 = _subidx(n1, 0.375)
    pos1, b1v = pos[i1], batch[i1]
    pos2, b2v = pos1[i2], b1v[i2]

    rel0, m0s, idx0 = _knn(pos, batch, pos, batch, 128)
    rel1, m1s, idx1, m1e = _knn(pos1, b1v, pos1, b1v, 128,
                                par_ids=i1, npar=n0)
    rel2, m2s, idx2, m2e = _knn(pos2, b2v, pos2, b2v, 192,
                                par_ids=i2, npar=n1)
    f0 = _pconv(rel0, Wp1_d0, Wp2_d0, Wf_d0)
    f1 = _pconv(rel1, Wp1_d1, Wp2_d1, Wf_d1[:32], m=m1e,
                xs=[(f0, Wf_d1[32:])])
    f2 = _pconv(rel2, Wp1_d2, Wp2_d2, Wf_d2[:32], m=m2e,
                xs=[(f1, Wf_d2[32:])])
    f3 = _pconv(rel2, Wp1_d3, Wp2_d3, Wf_d3[:32], m=m2s,
                xs=[(f2, Wf_d3[32:])])
    glob = _pool(f3, b2v, W1, b1, W2, b2)
    u0 = _pconv(rel2, Wp1_u0, Wp2_u0, Wf_u0[:256], m=m2s,
                xs=[(f3, Wf_u0[384:768]), (f3, Wf_u0[768:])],
                glob_piece=(b2v, glob, Wf_u0[256:384]))
    u1 = _pconv(rel2, Wp1_u1, Wp2_u1, Wf_u1[:256], m=m2s,
                xs=[(u0, Wf_u1[256:640]), (f2, Wf_u1[640:])])
    wi1 = _interp_w(pos1, b1v, pos2, b2v, 128)
    v1 = _matmul(wi1, u1)
    u2 = _pconv(rel1, Wp1_u2, Wp2_u2, Wf_u2[:256], m=m1s,
                xs=[(v1, Wf_u2[256:448]), (f1, Wf_u2[448:])])
    wi0 = _interp_w(pos, batch, pos1, b1v, 128)
    v0 = _matmul(wi0, u2)
    u3 = _pconv(rel0, Wp1_u3, Wp2_u3, Wf_u3[:256], m=m0s,
                xs=[(v0, Wf_u3[256:352]), (f0, Wf_u3[352:])])
    return _head(u3, Wh1, gamma, beta, Wh2, bh2)
